# baseline (speedup 1.0000x reference)
import functools

import jax
import jax.numpy as jnp
from jax import lax
from jax.experimental import pallas as pl
from jax.experimental.pallas import tpu as pltpu
from jax.experimental.pallas import tpu_sc as plsc

BATCH = 16384
DIM = 32
INPUT_ROWS = 1000000
NC = 2
NS = 16
NW = NC * NS
BPW = BATCH // NW
CH = 128
NCH = BPW // CH


def _sc_gather_body(i_hbm, j_hbm, wi_hbm, wj_hbm, bi_hbm, bj_hbm,
                    ri_hbm, rj_hbm, bri_hbm, brj_hbm,
                    idx_i, idx_j, rows_i, rows_j, bri_v, brj_v, sem):
  wid = lax.axis_index("s") * NC + lax.axis_index("c")
  base = wid * BPW
  for c in range(NCH):
    pltpu.sync_copy(i_hbm.at[pl.ds(base + c * CH, CH)], idx_i.at[c])
    pltpu.sync_copy(j_hbm.at[pl.ds(base + c * CH, CH)], idx_j.at[c])
  copies = []
  for c in range(NCH):
    sl = pl.ds(c * CH, CH)
    copies.append(pltpu.async_copy(wi_hbm.at[idx_i.at[c]], rows_i.at[sl], sem))
    copies.append(pltpu.async_copy(wj_hbm.at[idx_j.at[c]], rows_j.at[sl], sem))
    copies.append(pltpu.async_copy(bi_hbm.at[idx_i.at[c]], bri_v.at[sl], sem))
    copies.append(pltpu.async_copy(bj_hbm.at[idx_j.at[c]], brj_v.at[sl], sem))
  for cp in copies:
    cp.wait()
  out_sl = pl.ds(base, BPW)
  pltpu.sync_copy(rows_i, ri_hbm.at[out_sl])
  pltpu.sync_copy(rows_j, rj_hbm.at[out_sl])
  pltpu.sync_copy(bri_v, bri_hbm.at[out_sl])
  pltpu.sync_copy(brj_v, brj_hbm.at[out_sl])


@jax.jit
def _sc_gather(i, j, wi, wj, bi, bj):
  mesh = plsc.VectorSubcoreMesh(core_axis_name="c", subcore_axis_name="s")
  fn = functools.partial(
      pl.kernel, mesh=mesh,
      out_type=(
          jax.ShapeDtypeStruct((BATCH, DIM), jnp.float32),
          jax.ShapeDtypeStruct((BATCH, DIM), jnp.float32),
          jax.ShapeDtypeStruct((BATCH,), jnp.float32),
          jax.ShapeDtypeStruct((BATCH,), jnp.float32),
      ),
      scratch_types=[
          pltpu.VMEM((NCH, CH), jnp.int32),
          pltpu.VMEM((NCH, CH), jnp.int32),
          pltpu.VMEM((BPW, DIM), jnp.float32),
          pltpu.VMEM((BPW, DIM), jnp.float32),
          pltpu.VMEM((BPW,), jnp.float32),
          pltpu.VMEM((BPW,), jnp.float32),
          pltpu.SemaphoreType.DMA,
      ],
      compiler_params=pltpu.CompilerParams(use_tc_tiling_on_sc=False),
  )(_sc_gather_body)
  return fn(i, j, wi, wj, bi, bj)


TC_BLK = 2048


def _tc_dot_body(ri_ref, rj_ref, bri_ref, brj_ref, o_ref):
  prod = ri_ref[...] * rj_ref[...]
  o_ref[...] = jnp.sum(prod, axis=1) + bri_ref[...] + brj_ref[...]


@jax.jit
def _tc_dot(ri, rj, bri, brj):
  grid = (BATCH // TC_BLK,)
  return pl.pallas_call(
      _tc_dot_body,
      grid=grid,
      in_specs=[
          pl.BlockSpec((TC_BLK, DIM), lambda g: (g, 0)),
          pl.BlockSpec((TC_BLK, DIM), lambda g: (g, 0)),
          pl.BlockSpec((TC_BLK,), lambda g: (g,)),
          pl.BlockSpec((TC_BLK,), lambda g: (g,)),
      ],
      out_specs=pl.BlockSpec((TC_BLK,), lambda g: (g,)),
      out_shape=jax.ShapeDtypeStruct((BATCH,), jnp.float32),
  )(ri, rj, bri, brj)


def kernel(i, j, wi, wj, bi, bj):
  ri, rj, bri, brj = _sc_gather(i, j, wi, wj,
                                bi.reshape(INPUT_ROWS), bj.reshape(INPUT_ROWS))
  return _tc_dot(ri, rj, bri, brj)
